# baseline (speedup 1.0000x reference)
import functools

import jax
import jax.numpy as jnp
from jax import lax
from jax.experimental import pallas as pl
from jax.experimental.pallas import tpu as pltpu
from jax.experimental.pallas import tpu_sc as plsc

VOCAB = 1000000
POS = 512
D = 64
B = 4096
L = 200
N = B * L
NC = 2
NS = 16
NW = NC * NS
T = N // NW
C = 256
SUB = C // 128
K = T // C
NBUF = 3
IDXROWS = N // 128
EPS = 1e-5
BS = 1024


@functools.partial(
    pl.kernel,
    out_type=jax.ShapeDtypeStruct((N // 2, 2 * D), jnp.float32),
    mesh=plsc.VectorSubcoreMesh(core_axis_name="c", subcore_axis_name="s"),
    compiler_params=pltpu.CompilerParams(
        needs_layout_passes=False, use_tc_tiling_on_sc=False),
    scratch_types=(
        [pltpu.VMEM((SUB, 128), jnp.int32) for _ in range(NBUF)]
        + [pltpu.VMEM((SUB, 128), jnp.int32) for _ in range(NBUF)]
        + [pltpu.VMEM((SUB, 128), jnp.int32) for _ in range(NBUF)]
        + [pltpu.VMEM((C, D), jnp.float32) for _ in range(NBUF)]
        + [
            pltpu.SemaphoreType.DMA,
            pltpu.SemaphoreType.DMA,
            pltpu.SemaphoreType.DMA,
            pltpu.SemaphoreType.DMA,
        ]
    ),
)
def _gather_sum_kernel(widx_hbm, hidx_hbm, tidx_hbm, ww_hbm, wh_hbm, wt_hbm,
                       out_hbm,
                       wi0, wi1, wi2, hi0, hi1, hi2, ti0, ti1, ti2,
                       rows0, rows1, rows2,
                       isem, wsem, asem, osem):
    wi = [wi0, wi1, wi2]
    hi = [hi0, hi1, hi2]
    ti = [ti0, ti1, ti2]
    rows = [rows0, rows1, rows2]

    wid = lax.axis_index("s") * NC + lax.axis_index("c")
    idx_row0 = wid * (T // 128)
    tok0_w = wid * T

    def fire_idx(k, b):
        row0 = idx_row0 + k * SUB
        pltpu.async_copy(widx_hbm.at[pl.ds(row0, SUB)], wi[b], isem)
        pltpu.async_copy(hidx_hbm.at[pl.ds(row0, SUB)], hi[b], isem)
        pltpu.async_copy(tidx_hbm.at[pl.ds(row0, SUB)], ti[b], isem)

    def wait_idx(b):
        for ref in (wi[b], hi[b], ti[b]):
            pltpu.make_async_copy(widx_hbm.at[pl.ds(idx_row0, SUB)], ref,
                                  isem).wait()

    def fire_word(b):
        for i in range(SUB):
            pltpu.async_copy(ww_hbm.at[wi[b].at[i]],
                             rows[b].at[pl.ds(i * 128, 128)], wsem)

    def wait_word(b):
        for i in range(SUB):
            pltpu.make_async_copy(ww_hbm.at[wi[b].at[i]],
                                  rows[b].at[pl.ds(i * 128, 128)],
                                  wsem).wait()

    def fire_ht(b):
        for i in range(SUB):
            pltpu.async_copy(wh_hbm.at[hi[b].at[i]],
                             rows[b].at[pl.ds(i * 128, 128)], asem,
                             add=True)
            pltpu.async_copy(wt_hbm.at[ti[b].at[i]],
                             rows[b].at[pl.ds(i * 128, 128)], asem,
                             add=True)

    def wait_ht(b):
        for i in range(SUB):
            for _ in range(2):
                pltpu.make_async_copy(wh_hbm.at[hi[b].at[i]],
                                      rows[b].at[pl.ds(i * 128, 128)],
                                      asem).wait()

    half = wid // NS
    col0 = half * D
    orow_w = tok0_w - half * (N // 2)

    def fire_out(k, b):
        row0 = orow_w + k * C
        pltpu.async_copy(rows[b],
                         out_hbm.at[pl.ds(row0, C), pl.ds(col0, D)], osem)

    def wait_out(b):
        pltpu.make_async_copy(rows[b],
                              out_hbm.at[pl.ds(orow_w, C), pl.ds(col0, D)],
                              osem).wait()

    def iteration(k, p0, p1, p2):
        @pl.when(jnp.logical_and(k >= 1, k + 2 < K))
        def _():
            wait_out(p2)

        @pl.when(k + 2 < K)
        def _():
            fire_idx(k + 2, p2)

        @pl.when(k + 1 < K)
        def _():
            wait_word(p1)

        @pl.when(k + 2 < K)
        def _():
            wait_idx(p2)
            fire_word(p2)
        wait_ht(p0)

        @pl.when(k + 1 < K)
        def _():
            fire_ht(p1)
        fire_out(k, p0)

    fire_idx(0, 0)
    wait_idx(0)
    fire_word(0)
    wait_word(0)
    fire_ht(0)
    fire_idx(1, 1)
    wait_idx(1)
    fire_word(1)

    def body(k, carry):
        for p in range(NBUF):
            @pl.when(k % NBUF == p)
            def _(p=p):
                iteration(k, p, (p + 1) % NBUF, (p + 2) % NBUF)
        return carry

    lax.fori_loop(0, K, body, 0)

    for _ in range(NBUF):
        wait_out(0)


def _ln_body(x_ref, g_ref, b_ref, o_ref):
    h = pl.program_id(0)
    x = x_ref[...]
    v = jnp.where(h == 0, x[:, :D], x[:, D:])
    mean = jnp.mean(v, axis=1, keepdims=True)
    c = v - mean
    var = jnp.mean(c * c, axis=1, keepdims=True)
    o_ref[...] = c * lax.rsqrt(var + EPS) * g_ref[0, :] + b_ref[0, :]


_M = N // 2 // BS
_ln_kernel = pl.pallas_call(
    _ln_body,
    grid=(2, _M),
    in_specs=[
        pl.BlockSpec((BS, 2 * D), lambda h, i: (i, 0)),
        pl.BlockSpec((1, D), lambda h, i: (0, 0)),
        pl.BlockSpec((1, D), lambda h, i: (0, 0)),
    ],
    out_specs=pl.BlockSpec((BS, D), lambda h, i: (h * _M + i, 0)),
    out_shape=jax.ShapeDtypeStruct((N, D), jnp.float32),
)


def kernel(word, head, tail, W_word, W_head, W_tail, gamma, beta):
    wf = word.reshape(IDXROWS, 128)
    hf = head.reshape(IDXROWS, 128)
    tf = tail.reshape(IDXROWS, 128)
    sums = _gather_sum_kernel(wf, hf, tf, W_word, W_head, W_tail)
    out = _ln_kernel(sums, gamma.reshape(1, D), beta.reshape(1, D))
    return out.reshape(B, L, D)
